# baseline (speedup 1.0000x reference)
import functools

import jax
import jax.numpy as jnp
from jax import lax
from jax.experimental import pallas as pl
from jax.experimental.pallas import tpu as pltpu
from jax.experimental.pallas import tpu_sc as plsc

_F32 = jnp.float32

_pallas_call = pl.pallas_call


def _mm(a, b):
    return lax.dot_general(a, b, (((1,), (0,)), ((), ())),
                           preferred_element_type=_F32)


def _mm_t(a, b):
    return lax.dot_general(a, b, (((1,), (1,)), ((), ())),
                           preferred_element_type=_F32)


def _conv3x3_acc(pad_ref, w_ref):
    Bn, Hp, Wp, Ci = pad_ref.shape
    H, W = Hp - 2, Wp - 2
    Co = w_ref.shape[3]
    acc = jnp.zeros((Bn * H * W, Co), _F32)
    for dy in range(3):
        for dx in range(3):
            xs = pad_ref[:, dy:dy + H, dx:dx + W, :]
            acc = acc + _mm(xs.reshape(Bn * H * W, Ci), w_ref[dy, dx])
    return acc


def _conv3x3_same(xmap, w_ref, pad_ref):
    Bn, H, W, Ci = xmap.shape
    pad_ref[...] = jnp.zeros(pad_ref.shape, _F32)
    pad_ref[:, 1:H + 1, 1:W + 1, :] = xmap
    return _conv3x3_acc(pad_ref, w_ref)


def _conv3x3_valid(xmap, w_ref):
    Bn, H, W, Ci = xmap.shape
    Co = w_ref.shape[3]
    Ho, Wo = H - 2, W - 2
    acc = jnp.zeros((Bn * Ho * Wo, Co), _F32)
    for dy in range(3):
        for dx in range(3):
            xs = xmap[:, dy:dy + Ho, dx:dx + Wo, :]
            acc = acc + _mm(xs.reshape(Bn * Ho * Wo, Ci), w_ref[dy, dx])
    return acc


def _convT4x4_s2(xmap, w_ref, pad_ref):
    Bn, H, W, Ci = xmap.shape
    Co = w_ref.shape[3]
    pad_ref[...] = jnp.zeros(pad_ref.shape, _F32)
    pad_ref[:, 1:H + 1, 1:W + 1, :] = xmap
    taps = {0: ((0, -1), (2, 0)), 1: ((1, 0), (3, 1))}
    ys = []
    for a in (0, 1):
        row = []
        for c in (0, 1):
            acc = jnp.zeros((Bn * H * W, Co), _F32)
            for ky, sy in taps[a]:
                for kx, sx in taps[c]:
                    xs = pad_ref[:, 1 + sy:1 + sy + H, 1 + sx:1 + sx + W, :]
                    acc = acc + _mm(xs.reshape(Bn * H * W, Ci), w_ref[ky, kx])
            row.append(acc)
        ys.append(row)
    zrows = []
    for a in (0, 1):
        z = jnp.concatenate([ys[a][0].reshape(Bn * H * W, 1, Co),
                             ys[a][1].reshape(Bn * H * W, 1, Co)], axis=1)
        zrows.append(z.reshape(Bn * H, 1, 2 * W, Co))
    y = jnp.concatenate(zrows, axis=1)
    return y.reshape(Bn * 2 * H * 2 * W, Co)


def _vq_dist_argmin(zf, cb):
    N = zf.shape[0]
    K = cb.shape[0]
    c2 = jnp.sum(cb * cb, axis=1, keepdims=True)
    hi = c2.astype(jnp.bfloat16).astype(_F32)
    aug_z = jnp.concatenate([-2.0 * zf, jnp.ones((N, 2), _F32)], axis=1)
    aug_c = jnp.concatenate([cb, hi, c2 - hi], axis=1)
    d = _mm_t(aug_z, aug_c)
    m = jnp.min(d, axis=1, keepdims=True)
    ii = lax.broadcasted_iota(jnp.int32, d.shape, 1)
    idx = jnp.min(jnp.where(d <= m, ii, jnp.int32(K)), axis=1, keepdims=True)
    return idx


def _perplexity(idx, K):
    N = idx.shape[0]
    ii = lax.broadcasted_iota(jnp.int32, (N, K), 1)
    onehot = (idx == ii).astype(_F32)
    probs = jnp.sum(onehot, axis=0) / float(N)
    ent = -jnp.sum(probs * jnp.log(probs + 1e-10))
    return jnp.exp(ent)


def _bridge_body(lve_ref, txt_ref, qt_ref, wv_ref, wt_ref, wb_ref, wql_ref,
                 wqc_ref, cbt_ref, zeb_ref, zft_ref, idt_ref, perp_ref,
                 padb_ref, padq_ref):
    t = _mm(jnp.sum(txt_ref[...], axis=1) / 77.0, wt_ref[...])
    padb_ref[...] = jnp.zeros(padb_ref.shape, _F32)
    for b in range(4):
        vb = lax.dot_general(lve_ref[b], wv_ref[...],
                             (((0,), (0,)), ((), ())),
                             preferred_element_type=_F32)
        hb = jax.nn.relu(vb + t[b:b + 1, :])
        padb_ref[b, 1:17, 1:17, :] = hb.reshape(16, 16, 128)
    zeb_ref[...] = _conv3x3_acc(padb_ref, wb_ref)
    q = jax.nn.relu(_mm(qt_ref[...].reshape(256, 128), wql_ref[...]))
    zft = _conv3x3_same(q.reshape(4, 8, 8, 128), wqc_ref, padq_ref)
    zft_ref[...] = zft
    idx = _vq_dist_argmin(zft, cbt_ref[...])
    idt_ref[...] = jnp.min(idx, axis=1)
    perp_ref[...] = _perplexity(idx, 1024)[None, None]


def _bridge_call(lve3, txt, qt, wv, wt, wb, wql, wqc, cbt):
    return _pallas_call(
        _bridge_body,
        out_shape=[
            jax.ShapeDtypeStruct((1024, 128), _F32),
            jax.ShapeDtypeStruct((256, 128), _F32),
            jax.ShapeDtypeStruct((256,), jnp.int32),
            jax.ShapeDtypeStruct((1, 1), _F32),
        ],
        scratch_shapes=[
            pltpu.VMEM((4, 18, 18, 128), _F32),
            pltpu.VMEM((4, 10, 10, 128), _F32),
        ],
    )(lve3, txt, qt, wv, wt, wb, wql, wqc, cbt)


def _mid_body(zqt_ref, zft_ref, zeb_ref, wtc_ref, wdc_ref, wup_ref, cbb_ref,
              zfb_ref, idb_ref, zup_ref, losst_ref, perpb_ref,
              padt_ref, pad_ref):
    zqt = zqt_ref[...]
    m = zqt.reshape(4, 8, 8, 128)
    dact = jax.nn.relu(_convT4x4_s2(m, wtc_ref, padt_ref))
    zt2b = _conv3x3_same(dact.reshape(4, 16, 16, 256), wdc_ref, pad_ref)
    zfb = jnp.concatenate([zeb_ref[...], zt2b], axis=1)
    zfb_ref[...] = zfb
    idx = _vq_dist_argmin(zfb, cbb_ref[...])
    idb_ref[...] = jnp.min(idx, axis=1)
    perpb_ref[...] = _perplexity(idx, 1024)[None, None]
    zup_ref[...] = _convT4x4_s2(m, wup_ref, padt_ref)
    losst_ref[...] = (1.25 * jnp.mean((zqt - zft_ref[...]) ** 2))[None, None]


def _mid_call(zqt, zft, zeb, wtc, wdc, wup, cbb):
    return _pallas_call(
        _mid_body,
        out_shape=[
            jax.ShapeDtypeStruct((1024, 256), _F32),
            jax.ShapeDtypeStruct((1024,), jnp.int32),
            jax.ShapeDtypeStruct((1024, 128), _F32),
            jax.ShapeDtypeStruct((1, 1), _F32),
            jax.ShapeDtypeStruct((1, 1), _F32),
        ],
        scratch_shapes=[
            pltpu.VMEM((4, 10, 10, 128), _F32),
            pltpu.VMEM((4, 18, 18, 256), _F32),
        ],
    )(zqt, zft, zeb, wtc, wdc, wup, cbb)


def _final_body(zup_ref, zqb_ref, zfb_ref, losst_ref, wdn_ref, wres_ref,
                wbt_ref, wout_ref, xrec_ref, loss_ref,
                pad16_ref, pad30_ref):
    zqb = zqb_ref[...]
    zq = jnp.concatenate([zup_ref[...], zqb], axis=1)
    zd = _conv3x3_valid(zq.reshape(4, 16, 16, 384), wdn_ref)
    res = _conv3x3_same(zd.reshape(4, 14, 14, 384), wres_ref, pad16_ref)
    r = zd + jax.nn.relu(res)
    rup = jax.nn.relu(
        _convT4x4_s2(r.reshape(4, 14, 14, 384), wbt_ref, pad16_ref))
    xr = _conv3x3_same(rup.reshape(4, 28, 28, 384), wout_ref, pad30_ref)
    xrec_ref[...] = xr.reshape(4, 28, 28, 3)
    lb = 1.25 * jnp.mean((zqb - zfb_ref[...]) ** 2)
    loss_ref[...] = losst_ref[...] + lb


def _final_call(zup, zqb, zfb, losst, wdn, wres, wbt, wout):
    return _pallas_call(
        _final_body,
        out_shape=[
            jax.ShapeDtypeStruct((4, 28, 28, 3), _F32),
            jax.ShapeDtypeStruct((1, 1), _F32),
        ],
        scratch_shapes=[
            pltpu.VMEM((4, 16, 16, 384), _F32),
            pltpu.VMEM((4, 30, 30, 384), _F32),
        ],
    )(zup, zqb, zfb, losst, wdn, wres, wbt, wout)


def _sc_gather(table, idx):
    N = idx.shape[0]
    D = table.shape[1]
    info = plsc.get_sparse_core_info()
    nw = info.num_cores * info.num_subcores
    bpw = N // nw
    mesh = plsc.VectorSubcoreMesh(core_axis_name="c", subcore_axis_name="s")

    @functools.partial(
        pl.kernel, mesh=mesh,
        out_type=jax.ShapeDtypeStruct((N, D), _F32),
        scratch_types=[
            pltpu.VMEM((bpw,), jnp.int32),
            pltpu.VMEM((bpw, D), _F32),
            pltpu.SemaphoreType.DMA,
        ],
    )
    def gather(table_hbm, idx_hbm, out_hbm, idx_v, rows_v, sem):
        wid = lax.axis_index("s") * info.num_cores + lax.axis_index("c")
        base = wid * bpw
        pltpu.sync_copy(idx_hbm.at[pl.ds(base, bpw)], idx_v)
        pltpu.async_copy(table_hbm.at[idx_v], rows_v, sem).wait()
        pltpu.sync_copy(rows_v, out_hbm.at[pl.ds(base, bpw)])

    return gather(table, idx)


def kernel(local_visual_embeddings, text_embeddings, query_tokens, Wv, Wt,
           Wb_conv, Wq_lin, Wq_conv, cb_top, cb_bottom, dec_top_convT,
           dec_top_conv, up_convT, down_conv, dec_bot_res, dec_bot_convT,
           dec_bot_out):
    lve3 = local_visual_embeddings.reshape(4, 768, 256)
    wv = Wv.reshape(768, 128)

    zeb, zft, idt, perp_top = _bridge_call(
        lve3, text_embeddings, query_tokens, wv, Wt, Wb_conv, Wq_lin,
        Wq_conv, cb_top)
    zqt = _sc_gather(cb_top, idt)
    zfb, idb, zup, loss_top, perp_bottom = _mid_call(
        zqt, zft, zeb, dec_top_convT, dec_top_conv, up_convT, cb_bottom)
    zqb = _sc_gather(cb_bottom, idb)
    xr, loss = _final_call(
        zup, zqb, zfb, loss_top, down_conv, dec_bot_res, dec_bot_convT,
        dec_bot_out)

    x_recon = jnp.transpose(xr, (0, 3, 1, 2))
    return (x_recon, loss.reshape(()), perp_top.reshape(()),
            perp_bottom.reshape(()))


# speedup vs baseline: 1.0049x; 1.0049x over previous
import functools

import jax
import jax.numpy as jnp
from jax import lax
from jax.experimental import pallas as pl
from jax.experimental.pallas import tpu as pltpu
from jax.experimental.pallas import tpu_sc as plsc

_F32 = jnp.float32

_pallas_call = pl.pallas_call


def _mm(a, b):
    return lax.dot_general(a, b, (((1,), (0,)), ((), ())),
                           preferred_element_type=_F32)


def _mmb(a, b):
    return lax.dot_general(a.astype(jnp.bfloat16), b.astype(jnp.bfloat16),
                           (((1,), (0,)), ((), ())),
                           preferred_element_type=_F32)


def _mm_t(a, b):
    return lax.dot_general(a, b, (((1,), (1,)), ((), ())),
                           preferred_element_type=_F32)


def _conv3x3_acc(pad_ref, w_ref, mm=_mm):
    Bn, Hp, Wp, Ci = pad_ref.shape
    H, W = Hp - 2, Wp - 2
    Co = w_ref.shape[3]
    acc = jnp.zeros((Bn * H * W, Co), _F32)
    for dy in range(3):
        for dx in range(3):
            xs = pad_ref[:, dy:dy + H, dx:dx + W, :]
            acc = acc + mm(xs.reshape(Bn * H * W, Ci), w_ref[dy, dx])
    return acc


def _conv3x3_same(xmap, w_ref, pad_ref, mm=_mm):
    Bn, H, W, Ci = xmap.shape
    pad_ref[...] = jnp.zeros(pad_ref.shape, _F32)
    pad_ref[:, 1:H + 1, 1:W + 1, :] = xmap
    return _conv3x3_acc(pad_ref, w_ref, mm)


def _conv3x3_valid(xmap, w_ref, mm=_mm):
    Bn, H, W, Ci = xmap.shape
    Co = w_ref.shape[3]
    Ho, Wo = H - 2, W - 2
    acc = jnp.zeros((Bn * Ho * Wo, Co), _F32)
    for dy in range(3):
        for dx in range(3):
            xs = xmap[:, dy:dy + Ho, dx:dx + Wo, :]
            acc = acc + mm(xs.reshape(Bn * Ho * Wo, Ci), w_ref[dy, dx])
    return acc


def _convT4x4_s2(xmap, w_ref, pad_ref, mm=_mm):
    Bn, H, W, Ci = xmap.shape
    Co = w_ref.shape[3]
    pad_ref[...] = jnp.zeros(pad_ref.shape, _F32)
    pad_ref[:, 1:H + 1, 1:W + 1, :] = xmap
    taps = {0: ((0, -1), (2, 0)), 1: ((1, 0), (3, 1))}
    ys = []
    for a in (0, 1):
        row = []
        for c in (0, 1):
            acc = jnp.zeros((Bn * H * W, Co), _F32)
            for ky, sy in taps[a]:
                for kx, sx in taps[c]:
                    xs = pad_ref[:, 1 + sy:1 + sy + H, 1 + sx:1 + sx + W, :]
                    acc = acc + mm(xs.reshape(Bn * H * W, Ci), w_ref[ky, kx])
            row.append(acc)
        ys.append(row)
    zrows = []
    for a in (0, 1):
        z = jnp.concatenate([ys[a][0].reshape(Bn * H * W, 1, Co),
                             ys[a][1].reshape(Bn * H * W, 1, Co)], axis=1)
        zrows.append(z.reshape(Bn * H, 1, 2 * W, Co))
    y = jnp.concatenate(zrows, axis=1)
    return y.reshape(Bn * 2 * H * 2 * W, Co)


def _vq_dist_argmin(zf, cb):
    N = zf.shape[0]
    K = cb.shape[0]
    c2 = jnp.sum(cb * cb, axis=1, keepdims=True)
    hi = c2.astype(jnp.bfloat16).astype(_F32)
    aug_z = jnp.concatenate([-2.0 * zf, jnp.ones((N, 2), _F32)], axis=1)
    aug_c = jnp.concatenate([cb, hi, c2 - hi], axis=1)
    d = _mm_t(aug_z, aug_c)
    m = jnp.min(d, axis=1, keepdims=True)
    ii = lax.broadcasted_iota(jnp.int32, d.shape, 1)
    idx = jnp.min(jnp.where(d <= m, ii, jnp.int32(K)), axis=1, keepdims=True)
    return idx


def _perplexity(idx, K):
    N = idx.shape[0]
    ii = lax.broadcasted_iota(jnp.int32, (N, K), 1)
    onehot = (idx == ii).astype(_F32)
    probs = jnp.sum(onehot, axis=0) / float(N)
    ent = -jnp.sum(probs * jnp.log(probs + 1e-10))
    return jnp.exp(ent)


def _bridge_body(lve_ref, txt_ref, qt_ref, wv_ref, wt_ref, wb_ref, wql_ref,
                 wqc_ref, cbt_ref, zeb_ref, zft_ref, idt_ref, perp_ref,
                 padb_ref, padq_ref):
    t = _mm(jnp.sum(txt_ref[...], axis=1) / 77.0, wt_ref[...])
    padb_ref[...] = jnp.zeros(padb_ref.shape, _F32)
    for b in range(4):
        vb = lax.dot_general(lve_ref[b], wv_ref[...],
                             (((0,), (0,)), ((), ())),
                             preferred_element_type=_F32)
        hb = jax.nn.relu(vb + t[b:b + 1, :])
        padb_ref[b, 1:17, 1:17, :] = hb.reshape(16, 16, 128)
    zeb_ref[...] = _conv3x3_acc(padb_ref, wb_ref)
    q = jax.nn.relu(_mm(qt_ref[...].reshape(256, 128), wql_ref[...]))
    zft = _conv3x3_same(q.reshape(4, 8, 8, 128), wqc_ref, padq_ref)
    zft_ref[...] = zft
    idx = _vq_dist_argmin(zft, cbt_ref[...])
    idt_ref[...] = jnp.min(idx, axis=1)
    perp_ref[...] = _perplexity(idx, 1024)[None, None]


def _bridge_call(lve3, txt, qt, wv, wt, wb, wql, wqc, cbt):
    return _pallas_call(
        _bridge_body,
        out_shape=[
            jax.ShapeDtypeStruct((1024, 128), _F32),
            jax.ShapeDtypeStruct((256, 128), _F32),
            jax.ShapeDtypeStruct((256,), jnp.int32),
            jax.ShapeDtypeStruct((1, 1), _F32),
        ],
        scratch_shapes=[
            pltpu.VMEM((4, 18, 18, 128), _F32),
            pltpu.VMEM((4, 10, 10, 128), _F32),
        ],
    )(lve3, txt, qt, wv, wt, wb, wql, wqc, cbt)


def _mid_body(zqt_ref, zft_ref, zeb_ref, wtc_ref, wdc_ref, wup_ref, cbb_ref,
              zfb_ref, idb_ref, zup_ref, losst_ref, perpb_ref,
              padt_ref, pad_ref):
    zqt = zqt_ref[...]
    m = zqt.reshape(4, 8, 8, 128)
    dact = jax.nn.relu(_convT4x4_s2(m, wtc_ref, padt_ref))
    zt2b = _conv3x3_same(dact.reshape(4, 16, 16, 256), wdc_ref, pad_ref)
    zfb = jnp.concatenate([zeb_ref[...], zt2b], axis=1)
    zfb_ref[...] = zfb
    idx = _vq_dist_argmin(zfb, cbb_ref[...])
    idb_ref[...] = jnp.min(idx, axis=1)
    perpb_ref[...] = _perplexity(idx, 1024)[None, None]
    zup_ref[...] = _convT4x4_s2(m, wup_ref, padt_ref)
    losst_ref[...] = (1.25 * jnp.mean((zqt - zft_ref[...]) ** 2))[None, None]


def _mid_call(zqt, zft, zeb, wtc, wdc, wup, cbb):
    return _pallas_call(
        _mid_body,
        out_shape=[
            jax.ShapeDtypeStruct((1024, 256), _F32),
            jax.ShapeDtypeStruct((1024,), jnp.int32),
            jax.ShapeDtypeStruct((1024, 128), _F32),
            jax.ShapeDtypeStruct((1, 1), _F32),
            jax.ShapeDtypeStruct((1, 1), _F32),
        ],
        scratch_shapes=[
            pltpu.VMEM((4, 10, 10, 128), _F32),
            pltpu.VMEM((4, 18, 18, 256), _F32),
        ],
    )(zqt, zft, zeb, wtc, wdc, wup, cbb)


def _final_body(zup_ref, zqb_ref, zfb_ref, losst_ref, wdn_ref, wres_ref,
                wbt_ref, wout_ref, xrec_ref, loss_ref,
                pad16_ref, pad30_ref):
    zqb = zqb_ref[...]
    zq = jnp.concatenate([zup_ref[...], zqb], axis=1)
    zd = _conv3x3_valid(zq.reshape(4, 16, 16, 384), wdn_ref, _mmb)
    res = _conv3x3_same(zd.reshape(4, 14, 14, 384), wres_ref, pad16_ref, _mmb)
    r = zd + jax.nn.relu(res)
    rup = jax.nn.relu(
        _convT4x4_s2(r.reshape(4, 14, 14, 384), wbt_ref, pad16_ref, _mmb))
    pad30_ref[...] = jnp.zeros(pad30_ref.shape, _F32)
    pad30_ref[:, 1:29, 1:29, :] = rup.reshape(4, 28, 28, 384)
    xflat = pad30_ref[...].reshape(3600, 384)
    accp = jnp.zeros((3600, 3), _F32)
    for dy in range(3):
        for dx in range(3):
            off = (dy - 1) * 30 + (dx - 1)
            p = _mmb(xflat, wout_ref[dy, dx])
            ext = jnp.concatenate(
                [jnp.zeros((32, 3), _F32), p, jnp.zeros((32, 3), _F32)],
                axis=0)
            accp = accp + ext[32 + off:32 + off + 3600, :]
    xrec_ref[...] = accp.reshape(4, 30, 30, 3)[:, 1:29, 1:29, :]
    lb = 1.25 * jnp.mean((zqb - zfb_ref[...]) ** 2)
    loss_ref[...] = losst_ref[...] + lb


def _final_call(zup, zqb, zfb, losst, wdn, wres, wbt, wout):
    return _pallas_call(
        _final_body,
        out_shape=[
            jax.ShapeDtypeStruct((4, 28, 28, 3), _F32),
            jax.ShapeDtypeStruct((1, 1), _F32),
        ],
        scratch_shapes=[
            pltpu.VMEM((4, 16, 16, 384), _F32),
            pltpu.VMEM((4, 30, 30, 384), _F32),
        ],
    )(zup, zqb, zfb, losst, wdn, wres, wbt, wout)


def _sc_gather(table, idx):
    N = idx.shape[0]
    D = table.shape[1]
    info = plsc.get_sparse_core_info()
    nw = info.num_cores * info.num_subcores
    bpw = N // nw
    mesh = plsc.VectorSubcoreMesh(core_axis_name="c", subcore_axis_name="s")

    @functools.partial(
        pl.kernel, mesh=mesh,
        out_type=jax.ShapeDtypeStruct((N, D), _F32),
        scratch_types=[
            pltpu.VMEM((bpw,), jnp.int32),
            pltpu.VMEM((bpw, D), _F32),
            pltpu.SemaphoreType.DMA,
        ],
    )
    def gather(table_hbm, idx_hbm, out_hbm, idx_v, rows_v, sem):
        wid = lax.axis_index("s") * info.num_cores + lax.axis_index("c")
        base = wid * bpw
        pltpu.sync_copy(idx_hbm.at[pl.ds(base, bpw)], idx_v)
        pltpu.async_copy(table_hbm.at[idx_v], rows_v, sem).wait()
        pltpu.sync_copy(rows_v, out_hbm.at[pl.ds(base, bpw)])

    return gather(table, idx)


def kernel(local_visual_embeddings, text_embeddings, query_tokens, Wv, Wt,
           Wb_conv, Wq_lin, Wq_conv, cb_top, cb_bottom, dec_top_convT,
           dec_top_conv, up_convT, down_conv, dec_bot_res, dec_bot_convT,
           dec_bot_out):
    lve3 = local_visual_embeddings.reshape(4, 768, 256)
    wv = Wv.reshape(768, 128)

    zeb, zft, idt, perp_top = _bridge_call(
        lve3, text_embeddings, query_tokens, wv, Wt, Wb_conv, Wq_lin,
        Wq_conv, cb_top)
    zqt = _sc_gather(cb_top, idt)
    zfb, idb, zup, loss_top, perp_bottom = _mid_call(
        zqt, zft, zeb, dec_top_convT, dec_top_conv, up_convT, cb_bottom)
    zqb = _sc_gather(cb_bottom, idb)
    xr, loss = _final_call(
        zup, zqb, zfb, loss_top, down_conv, dec_bot_res, dec_bot_convT,
        dec_bot_out)

    x_recon = jnp.transpose(xr, (0, 3, 1, 2))
    return (x_recon, loss.reshape(()), perp_top.reshape(()),
            perp_bottom.reshape(()))
